# baseline (speedup 1.0000x reference)
import functools

import jax
import jax.numpy as jnp
from jax import lax
from jax.experimental import pallas as pl
from jax.experimental.pallas import tpu as pltpu
from jax.experimental.pallas import tpu_sc as plsc

HIDDEN = 128
NC = 2
NS = 16
NW = NC * NS
CHUNK = 80
NBUF = 10


def _make_gather(n_rows: int):
    assert n_rows % (NW * CHUNK) == 0
    b_per_w = n_rows // NW
    nchunks = b_per_w // CHUNK
    assert nchunks % NBUF == 0
    nrounds = nchunks // NBUF
    mesh = plsc.VectorSubcoreMesh(core_axis_name="c", subcore_axis_name="s")

    scratch = [pltpu.VMEM((nchunks, CHUNK), jnp.int32)]
    scratch += [pltpu.VMEM((CHUNK, HIDDEN), jnp.float32) for _ in range(NBUF)]
    scratch += [pltpu.SemaphoreType.DMA for _ in range(2 * NBUF)]

    @functools.partial(
        pl.kernel,
        mesh=mesh,
        out_type=jax.ShapeDtypeStruct((n_rows, HIDDEN), jnp.float32),
        scratch_types=scratch,
    )
    def gather_kernel(idx_hbm, table_hbm, out_hbm, idx_v, *bufs_and_sems):
        rows = bufs_and_sems[:NBUF]
        gsem = bufs_and_sems[NBUF:2 * NBUF]
        wsem = bufs_and_sems[2 * NBUF:]
        wid = lax.axis_index("s") * NC + lax.axis_index("c")
        base = wid * b_per_w
        pltpu.sync_copy(idx_hbm.at[wid], idx_v)

        def gather_cp(g, b):
            return pltpu.make_async_copy(table_hbm.at[idx_v.at[g]], rows[b], gsem[b])

        def write_cp(g, b):
            dst = out_hbm.at[pl.ds(base + g * CHUNK, CHUNK)]
            return pltpu.make_async_copy(rows[b], dst, wsem[b])

        for b in range(NBUF):
            gather_cp(b, b).start()

        LAG = 3

        def round_body(r, carry):
            g0 = r * NBUF
            for b in range(NBUF):
                gather_cp(g0 + b, b).wait()
                write_cp(g0 + b, b).start()
                if b >= LAG:
                    bb = b - LAG
                    write_cp(g0 + bb, bb).wait()
                    gather_cp(g0 + NBUF + bb, bb).start()
            for bb in range(NBUF - LAG, NBUF):
                write_cp(g0 + bb, bb).wait()
                gather_cp(g0 + NBUF + bb, bb).start()
            return carry

        lax.fori_loop(0, nrounds - 1, round_body, 0)

        g0 = (nrounds - 1) * NBUF
        for b in range(NBUF):
            gather_cp(g0 + b, b).wait()
            write_cp(g0 + b, b).start()
        for b in range(NBUF):
            write_cp(g0 + b, b).wait()

    return gather_kernel


def kernel(input_ids, table):
    batch, seq = input_ids.shape
    n_rows = batch * seq
    idx = input_ids.reshape(NW, n_rows // (NW * CHUNK), CHUNK).astype(jnp.int32)
    out = _make_gather(n_rows)(idx, table)
    return out.reshape(batch, seq, HIDDEN)


# speedup vs baseline: 1.0013x; 1.0013x over previous
import functools

import jax
import jax.numpy as jnp
from jax import lax
from jax.experimental import pallas as pl
from jax.experimental.pallas import tpu as pltpu
from jax.experimental.pallas import tpu_sc as plsc

HIDDEN = 128
NC = 2
NS = 16
NW = NC * NS
CHUNK = 80
NBUF = 10


def _make_gather(n_rows: int):
    assert n_rows % (NW * CHUNK) == 0
    b_per_w = n_rows // NW
    nchunks = b_per_w // CHUNK
    assert nchunks % NBUF == 0
    nrounds = nchunks // NBUF
    mesh = plsc.VectorSubcoreMesh(core_axis_name="c", subcore_axis_name="s")

    scratch = [pltpu.VMEM((nchunks, CHUNK), jnp.int32)]
    scratch += [pltpu.VMEM((NBUF * CHUNK, HIDDEN), jnp.float32)]
    scratch += [pltpu.SemaphoreType.DMA for _ in range(NBUF + NBUF // 2)]

    @functools.partial(
        pl.kernel,
        mesh=mesh,
        out_type=jax.ShapeDtypeStruct((n_rows, HIDDEN), jnp.float32),
        scratch_types=scratch,
    )
    def gather_kernel(idx_hbm, table_hbm, out_hbm, idx_v, *bufs_and_sems):
        rows = bufs_and_sems[0]
        gsem = bufs_and_sems[1:1 + NBUF]
        wsem = bufs_and_sems[1 + NBUF:]
        wid = lax.axis_index("s") * NC + lax.axis_index("c")
        base = wid * b_per_w
        pltpu.sync_copy(idx_hbm.at[wid], idx_v)

        def gather_cp(g, b):
            return pltpu.make_async_copy(
                table_hbm.at[idx_v.at[g]],
                rows.at[pl.ds(b * CHUNK, CHUNK)], gsem[b])

        def write_cp(g, p):
            dst = out_hbm.at[pl.ds(base + g * CHUNK, 2 * CHUNK)]
            return pltpu.make_async_copy(
                rows.at[pl.ds(2 * p * CHUNK, 2 * CHUNK)], dst, wsem[p])

        for b in range(NBUF):
            gather_cp(b, b).start()

        NPAIR = NBUF // 2
        PLAG = 1

        def round_body(r, carry):
            g0 = r * NBUF
            for p in range(NPAIR):
                gather_cp(g0 + 2 * p, 2 * p).wait()
                gather_cp(g0 + 2 * p + 1, 2 * p + 1).wait()
                write_cp(g0 + 2 * p, p).start()
                if p >= PLAG:
                    pp = p - PLAG
                    write_cp(g0 + 2 * pp, pp).wait()
                    gather_cp(g0 + NBUF + 2 * pp, 2 * pp).start()
                    gather_cp(g0 + NBUF + 2 * pp + 1, 2 * pp + 1).start()
            for pp in range(NPAIR - PLAG, NPAIR):
                write_cp(g0 + 2 * pp, pp).wait()
                gather_cp(g0 + NBUF + 2 * pp, 2 * pp).start()
                gather_cp(g0 + NBUF + 2 * pp + 1, 2 * pp + 1).start()
            return carry

        lax.fori_loop(0, nrounds - 1, round_body, 0)

        g0 = (nrounds - 1) * NBUF
        for p in range(NPAIR):
            gather_cp(g0 + 2 * p, 2 * p).wait()
            gather_cp(g0 + 2 * p + 1, 2 * p + 1).wait()
            write_cp(g0 + 2 * p, p).start()
        for p in range(NPAIR):
            write_cp(g0 + 2 * p, p).wait()

    return gather_kernel


def kernel(input_ids, table):
    batch, seq = input_ids.shape
    n_rows = batch * seq
    idx = input_ids.reshape(NW, n_rows // (NW * CHUNK), CHUNK).astype(jnp.int32)
    out = _make_gather(n_rows)(idx, table)
    return out.reshape(batch, seq, HIDDEN)
